# baseline (speedup 1.0000x reference)
import jax
import jax.numpy as jnp
from jax import lax
from jax.experimental import pallas as pl
from jax.experimental.pallas import tpu as pltpu
from jax.experimental.pallas import tpu_sc as plsc

N = 32768
D = 32
NSEG = 16
NW = 16
CHUNK = N // NW
L = 16
G = CHUNK // L
NEG = float("-inf")

PACK = 4
MROWS = N // PACK
MK = D * PACK


def _tc_matvec(x_ref, w_ref, u_ref):
    u_ref[...] = lax.dot_general(
        x_ref[...], w_ref[...], (((1,), (0,)), ((), ())),
        preferred_element_type=jnp.float32) + 2.0


def _sc_softmax(u_hbm, ids_hbm, out_hbm, ub, idsb, lrb, allb, gb, ob, tb,
                shmax, shsum):
    w = lax.axis_index("s")
    base = w * CHUNK
    pltpu.sync_copy(u_hbm.at[pl.ds(base, CHUNK)], ub)
    pltpu.sync_copy(ids_hbm.at[pl.ds(base, CHUNK)], idsb.at[pl.ds(0, CHUNK)])

    lanes = lax.iota(jnp.int32, L)
    idsb[pl.ds(CHUNK, L)] = jnp.full((L,), -1, jnp.int32)

    lrb[...] = jnp.full((L,), NEG, jnp.float32)
    lo = jnp.min(idsb[pl.ds(0, L)])
    hi = jnp.max(idsb[pl.ds(CHUNK - L, L)])

    def seg_max(s, carry):
        def g_body(g, maccs):
            new = []
            for k in range(4):
                sl = pl.ds((4 * g + k) * L, L)
                new.append(jnp.maximum(
                    maccs[k], jnp.where(idsb[sl] == s, ub[sl], NEG)))
            return tuple(new)
        init = tuple(jnp.full((L,), NEG, jnp.float32) for _ in range(4))
        maccs = lax.fori_loop(0, G // 4, g_body, init)
        m = jnp.max(jnp.maximum(jnp.maximum(maccs[0], maccs[1]),
                                jnp.maximum(maccs[2], maccs[3])))
        plsc.store_scatter(lrb, [jnp.full((L,), s, jnp.int32)],
                           jnp.full((L,), m, jnp.float32), mask=lanes == 0)
        return carry

    lax.fori_loop(lo, hi + 1, seg_max, 0)

    pltpu.sync_copy(lrb, shmax.at[pl.ds(w * NSEG, NSEG)])
    plsc.subcore_barrier()
    pltpu.sync_copy(shmax, allb)
    gm = allb[pl.ds(0, L)]
    for r in range(1, NW):
        gm = jnp.maximum(gm, allb[pl.ds(r * L, L)])
    gb[...] = gm

    lrb[...] = jnp.zeros((L,), jnp.float32)

    def pass_b(g, offs):
        sl = pl.ds(g * L, L)
        idsv = idsb[sl]
        mseg = plsc.load_gather(gb, [idsv])
        ev = jnp.exp(ub[sl] - mseg)
        ub[sl] = ev
        t = plsc.cumsum(ev) + offs
        bmask = idsv != idsb[pl.ds(g * L + 1, L)]
        plsc.store_scatter(lrb, [idsv], t, mask=bmask)
        return offs + jnp.sum(ev)

    lax.fori_loop(0, G, pass_b, jnp.zeros((L,), jnp.float32))

    v = lrb[...]
    fwd = plsc.cummax(v)
    tb[pl.ds(0, L)] = fwd
    prev = plsc.load_gather(tb, [jnp.maximum(lanes - 1, 0)])
    prev = jnp.where(lanes == 0, 0.0, prev)
    lrb[...] = jnp.where(v == 0.0, 0.0, v - prev)

    pltpu.sync_copy(lrb, shsum.at[pl.ds(w * NSEG, NSEG)])
    plsc.subcore_barrier()
    pltpu.sync_copy(shsum, allb)
    gs = allb[pl.ds(0, L)]
    for r in range(1, NW):
        gs = gs + allb[pl.ds(r * L, L)]
    gb[...] = 1.0 / gs

    @plsc.parallel_loop(0, G, unroll=4)
    def pass_c(g):
        sl = pl.ds(g * L, L)
        rseg = plsc.load_gather(gb, [idsb[sl]])
        ob[sl] = ub[sl] * rseg

    pltpu.sync_copy(ob, out_hbm.at[pl.ds(base, CHUNK)])


def kernel(x, ids, W):
    ids32 = ids.astype(jnp.int32)
    xr = x.reshape(MROWS, MK)
    wvals = W.reshape(-1)[jnp.arange(MK) % D]
    blk = (jnp.arange(MK) // D)[:, None] == jnp.arange(PACK)[None, :]
    w4 = jnp.where(blk, wvals[:, None], 0.0).astype(jnp.float32)

    u4 = pl.pallas_call(
        _tc_matvec,
        grid=(8,),
        in_specs=[pl.BlockSpec((MROWS // 8, MK), lambda i: (i, 0)),
                  pl.BlockSpec((MK, PACK), lambda i: (0, 0))],
        out_specs=pl.BlockSpec((MROWS // 8, PACK), lambda i: (i, 0)),
        out_shape=jax.ShapeDtypeStruct((MROWS, PACK), jnp.float32),
    )(xr, w4)
    u = u4.reshape(N)

    mesh = plsc.VectorSubcoreMesh(
        core_axis_name="c", subcore_axis_name="s", num_cores=1,
        num_subcores=NW)
    soft = pl.kernel(
        _sc_softmax,
        out_type=jax.ShapeDtypeStruct((N,), jnp.float32),
        mesh=mesh,
        compiler_params=pltpu.CompilerParams(
            needs_layout_passes=False, disable_bounds_checks=True),
        scratch_types=[
            pltpu.VMEM((CHUNK,), jnp.float32),
            pltpu.VMEM((CHUNK + L,), jnp.int32),
            pltpu.VMEM((NSEG,), jnp.float32),
            pltpu.VMEM((NW * NSEG,), jnp.float32),
            pltpu.VMEM((NSEG,), jnp.float32),
            pltpu.VMEM((CHUNK,), jnp.float32),
            pltpu.VMEM((L,), jnp.float32),
            pltpu.VMEM_SHARED((NW * NSEG,), jnp.float32),
            pltpu.VMEM_SHARED((NW * NSEG,), jnp.float32),
        ],
    )(u, ids32)
    return soft[:, None]


# speedup vs baseline: 1.1921x; 1.1921x over previous
import jax
import jax.numpy as jnp
from jax import lax
from jax.experimental import pallas as pl
from jax.experimental.pallas import tpu as pltpu
from jax.experimental.pallas import tpu_sc as plsc

N = 32768
D = 32
NSEG = 16
NW = 16
CHUNK = N // NW
L = 16
G = CHUNK // L
NEG = float("-inf")

PACK = 4
MROWS = N // PACK
MK = D * PACK


def _tc_matvec(x_ref, w_ref, u_ref):
    u_ref[...] = jnp.sum(x_ref[...] * w_ref[...], axis=1) + 2.0


def _sc_softmax(u_hbm, ids_hbm, out_hbm, ub, idsb, lrb, allb, gb, ob, tb,
                shmax, shsum):
    w = lax.axis_index("s")
    base = w * CHUNK
    pltpu.sync_copy(u_hbm.at[pl.ds(base, CHUNK)], ub)
    pltpu.sync_copy(ids_hbm.at[pl.ds(base, CHUNK)], idsb.at[pl.ds(0, CHUNK)])

    lanes = lax.iota(jnp.int32, L)
    idsb[pl.ds(CHUNK, L)] = jnp.full((L,), -1, jnp.int32)

    lrb[...] = jnp.full((L,), NEG, jnp.float32)
    lo = jnp.min(idsb[pl.ds(0, L)])
    hi = jnp.max(idsb[pl.ds(CHUNK - L, L)])

    def seg_max(s, carry):
        def g_body(g, maccs):
            new = []
            for k in range(4):
                sl = pl.ds((4 * g + k) * L, L)
                new.append(jnp.maximum(
                    maccs[k], jnp.where(idsb[sl] == s, ub[sl], NEG)))
            return tuple(new)
        init = tuple(jnp.full((L,), NEG, jnp.float32) for _ in range(4))
        maccs = lax.fori_loop(0, G // 4, g_body, init)
        m = jnp.max(jnp.maximum(jnp.maximum(maccs[0], maccs[1]),
                                jnp.maximum(maccs[2], maccs[3])))
        plsc.store_scatter(lrb, [jnp.full((L,), s, jnp.int32)],
                           jnp.full((L,), m, jnp.float32), mask=lanes == 0)
        return carry

    lax.fori_loop(lo, hi + 1, seg_max, 0)

    pltpu.sync_copy(lrb, shmax.at[pl.ds(w * NSEG, NSEG)])
    plsc.subcore_barrier()
    pltpu.sync_copy(shmax, allb)
    gm = allb[pl.ds(0, L)]
    for r in range(1, NW):
        gm = jnp.maximum(gm, allb[pl.ds(r * L, L)])
    gb[...] = gm

    lrb[...] = jnp.zeros((L,), jnp.float32)

    def pass_b(g, offs):
        sl = pl.ds(g * L, L)
        idsv = idsb[sl]
        mseg = plsc.load_gather(gb, [idsv])
        ev = jnp.exp(ub[sl] - mseg)
        ub[sl] = ev
        t = plsc.cumsum(ev) + offs
        bmask = idsv != idsb[pl.ds(g * L + 1, L)]
        plsc.store_scatter(lrb, [idsv], t, mask=bmask)
        return offs + jnp.sum(ev)

    lax.fori_loop(0, G, pass_b, jnp.zeros((L,), jnp.float32))

    v = lrb[...]
    fwd = plsc.cummax(v)
    tb[pl.ds(0, L)] = fwd
    prev = plsc.load_gather(tb, [jnp.maximum(lanes - 1, 0)])
    prev = jnp.where(lanes == 0, 0.0, prev)
    lrb[...] = jnp.where(v == 0.0, 0.0, v - prev)

    pltpu.sync_copy(lrb, shsum.at[pl.ds(w * NSEG, NSEG)])
    plsc.subcore_barrier()
    pltpu.sync_copy(shsum, allb)
    gs = allb[pl.ds(0, L)]
    for r in range(1, NW):
        gs = gs + allb[pl.ds(r * L, L)]
    gb[...] = 1.0 / gs

    @plsc.parallel_loop(0, G, unroll=4)
    def pass_c(g):
        sl = pl.ds(g * L, L)
        rseg = plsc.load_gather(gb, [idsb[sl]])
        ob[sl] = ub[sl] * rseg

    pltpu.sync_copy(ob, out_hbm.at[pl.ds(base, CHUNK)])


def kernel(x, ids, W):
    ids32 = ids.astype(jnp.int32)

    u = pl.pallas_call(
        _tc_matvec,
        grid=(8,),
        in_specs=[pl.BlockSpec((N // 8, D), lambda i: (i, 0)),
                  pl.BlockSpec((1, D), lambda i: (0, 0))],
        out_specs=pl.BlockSpec((N // 8,), lambda i: (i,)),
        out_shape=jax.ShapeDtypeStruct((N,), jnp.float32),
    )(x, W)

    mesh = plsc.VectorSubcoreMesh(
        core_axis_name="c", subcore_axis_name="s", num_cores=1,
        num_subcores=NW)
    soft = pl.kernel(
        _sc_softmax,
        out_type=jax.ShapeDtypeStruct((N,), jnp.float32),
        mesh=mesh,
        compiler_params=pltpu.CompilerParams(
            needs_layout_passes=False, disable_bounds_checks=True),
        scratch_types=[
            pltpu.VMEM((CHUNK,), jnp.float32),
            pltpu.VMEM((CHUNK + L,), jnp.int32),
            pltpu.VMEM((NSEG,), jnp.float32),
            pltpu.VMEM((NW * NSEG,), jnp.float32),
            pltpu.VMEM((NSEG,), jnp.float32),
            pltpu.VMEM((CHUNK,), jnp.float32),
            pltpu.VMEM((L,), jnp.float32),
            pltpu.VMEM_SHARED((NW * NSEG,), jnp.float32),
            pltpu.VMEM_SHARED((NW * NSEG,), jnp.float32),
        ],
    )(u, ids32)
    return soft[:, None]


# speedup vs baseline: 2.0745x; 1.7402x over previous
import jax
import jax.numpy as jnp
from jax import lax
from jax.experimental import pallas as pl
from jax.experimental.pallas import tpu as pltpu
from jax.experimental.pallas import tpu_sc as plsc

N = 32768
D = 32
NSEG = 16
NW = 16
CHUNK = N // NW
L = 16
G = CHUNK // L
NEG = float("-inf")

PACK = 4
MROWS = N // PACK
MK = D * PACK


def _tc_matvec(xt_ref, w_ref, u_ref):
    u_ref[...] = jnp.sum(xt_ref[...] * w_ref[...], axis=0) + 2.0


def _sc_softmax(u_hbm, ids_hbm, out_hbm, ub, idsb, lrb, allb, gb, ob, tb,
                shmax, shsum):
    w = lax.axis_index("s")
    base = w * CHUNK
    pltpu.sync_copy(u_hbm.at[pl.ds(base, CHUNK)], ub)
    pltpu.sync_copy(ids_hbm.at[pl.ds(base, CHUNK)], idsb.at[pl.ds(0, CHUNK)])

    lanes = lax.iota(jnp.int32, L)
    idsb[pl.ds(CHUNK, L)] = jnp.full((L,), -1, jnp.int32)

    lrb[...] = jnp.full((L,), NEG, jnp.float32)
    lo = jnp.min(idsb[pl.ds(0, L)])
    hi = jnp.max(idsb[pl.ds(CHUNK - L, L)])

    def seg_max(s, carry):
        def g_body(g, maccs):
            new = []
            for k in range(4):
                sl = pl.ds((4 * g + k) * L, L)
                new.append(jnp.maximum(
                    maccs[k], jnp.where(idsb[sl] == s, ub[sl], NEG)))
            return tuple(new)
        init = tuple(jnp.full((L,), NEG, jnp.float32) for _ in range(4))
        maccs = lax.fori_loop(0, G // 4, g_body, init)
        m = jnp.max(jnp.maximum(jnp.maximum(maccs[0], maccs[1]),
                                jnp.maximum(maccs[2], maccs[3])))
        plsc.store_scatter(lrb, [jnp.full((L,), s, jnp.int32)],
                           jnp.full((L,), m, jnp.float32), mask=lanes == 0)
        return carry

    lax.fori_loop(lo, hi + 1, seg_max, 0)

    pltpu.sync_copy(lrb, shmax.at[pl.ds(w * NSEG, NSEG)])
    plsc.subcore_barrier()
    pltpu.sync_copy(shmax, allb)
    gm = allb[pl.ds(0, L)]
    for r in range(1, NW):
        gm = jnp.maximum(gm, allb[pl.ds(r * L, L)])
    gb[...] = gm

    lrb[...] = jnp.zeros((L,), jnp.float32)

    def pass_b(g, offs):
        sl = pl.ds(g * L, L)
        idsv = idsb[sl]
        mseg = plsc.load_gather(gb, [idsv])
        ev = jnp.exp(ub[sl] - mseg)
        ub[sl] = ev
        t = plsc.cumsum(ev) + offs
        bmask = idsv != idsb[pl.ds(g * L + 1, L)]
        plsc.store_scatter(lrb, [idsv], t, mask=bmask)
        return offs + jnp.sum(ev)

    lax.fori_loop(0, G, pass_b, jnp.zeros((L,), jnp.float32))

    v = lrb[...]
    fwd = plsc.cummax(v)
    tb[pl.ds(0, L)] = fwd
    prev = plsc.load_gather(tb, [jnp.maximum(lanes - 1, 0)])
    prev = jnp.where(lanes == 0, 0.0, prev)
    lrb[...] = jnp.where(v == 0.0, 0.0, v - prev)

    pltpu.sync_copy(lrb, shsum.at[pl.ds(w * NSEG, NSEG)])
    plsc.subcore_barrier()
    pltpu.sync_copy(shsum, allb)
    gs = allb[pl.ds(0, L)]
    for r in range(1, NW):
        gs = gs + allb[pl.ds(r * L, L)]
    gb[...] = 1.0 / gs

    @plsc.parallel_loop(0, G, unroll=4)
    def pass_c(g):
        sl = pl.ds(g * L, L)
        rseg = plsc.load_gather(gb, [idsb[sl]])
        ob[sl] = ub[sl] * rseg

    pltpu.sync_copy(ob, out_hbm.at[pl.ds(base, CHUNK)])


def kernel(x, ids, W):
    ids32 = ids.astype(jnp.int32)

    u = pl.pallas_call(
        _tc_matvec,
        grid=(8,),
        in_specs=[pl.BlockSpec((D, N // 8), lambda i: (0, i)),
                  pl.BlockSpec((D, 1), lambda i: (0, 0))],
        out_specs=pl.BlockSpec((N // 8,), lambda i: (i,)),
        out_shape=jax.ShapeDtypeStruct((N,), jnp.float32),
    )(x.T, W.T)

    mesh = plsc.VectorSubcoreMesh(
        core_axis_name="c", subcore_axis_name="s", num_cores=1,
        num_subcores=NW)
    soft = pl.kernel(
        _sc_softmax,
        out_type=jax.ShapeDtypeStruct((N,), jnp.float32),
        mesh=mesh,
        compiler_params=pltpu.CompilerParams(
            needs_layout_passes=False, disable_bounds_checks=True),
        scratch_types=[
            pltpu.VMEM((CHUNK,), jnp.float32),
            pltpu.VMEM((CHUNK + L,), jnp.int32),
            pltpu.VMEM((NSEG,), jnp.float32),
            pltpu.VMEM((NW * NSEG,), jnp.float32),
            pltpu.VMEM((NSEG,), jnp.float32),
            pltpu.VMEM((CHUNK,), jnp.float32),
            pltpu.VMEM((L,), jnp.float32),
            pltpu.VMEM_SHARED((NW * NSEG,), jnp.float32),
            pltpu.VMEM_SHARED((NW * NSEG,), jnp.float32),
        ],
    )(u, ids32)
    return soft[:, None]


# speedup vs baseline: 2.4201x; 1.1666x over previous
import jax
import jax.numpy as jnp
from jax import lax
from jax.experimental import pallas as pl
from jax.experimental.pallas import tpu as pltpu
from jax.experimental.pallas import tpu_sc as plsc

N = 32768
D = 32
NSEG = 16
NW = 16
CHUNK = N // NW
L = 16
G = CHUNK // L
NEG = float("-inf")

PACK = 4
MROWS = N // PACK
MK = D * PACK


def _tc_matvec(xt_ref, w_ref, u_ref):
    u_ref[...] = jnp.sum(xt_ref[...] * w_ref[...].T, axis=0) + 2.0


def _sc_softmax(u_hbm, ids_hbm, out_hbm, ub, idsb, lrb, allb, gb, ob, tb,
                shmax, shsum):
    w = lax.axis_index("s")
    base = w * CHUNK
    pltpu.sync_copy(u_hbm.at[pl.ds(base, CHUNK)], ub)
    pltpu.sync_copy(ids_hbm.at[pl.ds(base, CHUNK)], idsb.at[pl.ds(0, CHUNK)])

    lanes = lax.iota(jnp.int32, L)
    idsb[pl.ds(CHUNK, L)] = jnp.full((L,), -1, jnp.int32)

    lrb[...] = jnp.full((L,), NEG, jnp.float32)
    lo = jnp.min(idsb[pl.ds(0, L)])
    hi = jnp.max(idsb[pl.ds(CHUNK - L, L)])

    def seg_max(s, carry):
        def g_body(g, maccs):
            new = []
            for k in range(4):
                sl = pl.ds((4 * g + k) * L, L)
                new.append(jnp.maximum(
                    maccs[k], jnp.where(idsb[sl] == s, ub[sl], NEG)))
            return tuple(new)
        init = tuple(jnp.full((L,), NEG, jnp.float32) for _ in range(4))
        maccs = lax.fori_loop(0, G // 4, g_body, init)
        m = jnp.max(jnp.maximum(jnp.maximum(maccs[0], maccs[1]),
                                jnp.maximum(maccs[2], maccs[3])))
        plsc.store_scatter(lrb, [jnp.full((L,), s, jnp.int32)],
                           jnp.full((L,), m, jnp.float32), mask=lanes == 0)
        return carry

    lax.fori_loop(lo, hi + 1, seg_max, 0)

    pltpu.sync_copy(lrb, shmax.at[pl.ds(w * NSEG, NSEG)])
    plsc.subcore_barrier()
    pltpu.sync_copy(shmax, allb)
    gm = allb[pl.ds(0, L)]
    for r in range(1, NW):
        gm = jnp.maximum(gm, allb[pl.ds(r * L, L)])
    gb[...] = gm

    lrb[...] = jnp.zeros((L,), jnp.float32)

    def pass_b(g, offs):
        sl = pl.ds(g * L, L)
        idsv = idsb[sl]
        mseg = plsc.load_gather(gb, [idsv])
        ev = jnp.exp(ub[sl] - mseg)
        ub[sl] = ev
        t = plsc.cumsum(ev) + offs
        bmask = idsv != idsb[pl.ds(g * L + 1, L)]
        plsc.store_scatter(lrb, [idsv], t, mask=bmask)
        return offs + jnp.sum(ev)

    lax.fori_loop(0, G, pass_b, jnp.zeros((L,), jnp.float32))

    v = lrb[...]
    fwd = plsc.cummax(v)
    tb[pl.ds(0, L)] = fwd
    prev = plsc.load_gather(tb, [jnp.maximum(lanes - 1, 0)])
    prev = jnp.where(lanes == 0, 0.0, prev)
    lrb[...] = jnp.where(v == 0.0, 0.0, v - prev)

    pltpu.sync_copy(lrb, shsum.at[pl.ds(w * NSEG, NSEG)])
    plsc.subcore_barrier()
    pltpu.sync_copy(shsum, allb)
    gs = allb[pl.ds(0, L)]
    for r in range(1, NW):
        gs = gs + allb[pl.ds(r * L, L)]
    gb[...] = 1.0 / gs

    @plsc.parallel_loop(0, G, unroll=4)
    def pass_c(g):
        sl = pl.ds(g * L, L)
        rseg = plsc.load_gather(gb, [idsb[sl]])
        ob[sl] = ub[sl] * rseg

    pltpu.sync_copy(ob, out_hbm.at[pl.ds(base, CHUNK)])


def kernel(x, ids, W):
    ids32 = ids.astype(jnp.int32)

    u = pl.pallas_call(
        _tc_matvec,
        grid=(2,),
        in_specs=[pl.BlockSpec((D, N // 2), lambda i: (0, i)),
                  pl.BlockSpec((1, D), lambda i: (0, 0))],
        out_specs=pl.BlockSpec((N // 2,), lambda i: (i,)),
        out_shape=jax.ShapeDtypeStruct((N,), jnp.float32),
    )(x.T, W)

    mesh = plsc.VectorSubcoreMesh(
        core_axis_name="c", subcore_axis_name="s", num_cores=1,
        num_subcores=NW)
    soft = pl.kernel(
        _sc_softmax,
        out_type=jax.ShapeDtypeStruct((N,), jnp.float32),
        mesh=mesh,
        compiler_params=pltpu.CompilerParams(
            needs_layout_passes=False, disable_bounds_checks=True),
        scratch_types=[
            pltpu.VMEM((CHUNK,), jnp.float32),
            pltpu.VMEM((CHUNK + L,), jnp.int32),
            pltpu.VMEM((NSEG,), jnp.float32),
            pltpu.VMEM((NW * NSEG,), jnp.float32),
            pltpu.VMEM((NSEG,), jnp.float32),
            pltpu.VMEM((CHUNK,), jnp.float32),
            pltpu.VMEM((L,), jnp.float32),
            pltpu.VMEM_SHARED((NW * NSEG,), jnp.float32),
            pltpu.VMEM_SHARED((NW * NSEG,), jnp.float32),
        ],
    )(u, ids32)
    return soft[:, None]


# speedup vs baseline: 2.6494x; 1.0948x over previous
import jax
import jax.numpy as jnp
from jax import lax
from jax.experimental import pallas as pl
from jax.experimental.pallas import tpu as pltpu
from jax.experimental.pallas import tpu_sc as plsc

N = 32768
D = 32
NSEG = 16
NW = 16
CHUNK = N // NW
L = 16
G = CHUNK // L
NEG = float("-inf")

PACK = 4
MROWS = N // PACK
MK = D * PACK


def _tc_matvec(xt_ref, w_ref, u_ref):
    u_ref[...] = jnp.sum(xt_ref[...] * w_ref[...].T, axis=0) + 2.0


def _sc_softmax(u_hbm, ids_hbm, out_hbm, ub, idsb, lrb, allb, gb, ob, tb,
                totb, offsb, shmax, shsum):
    w = lax.axis_index("s")
    base = w * CHUNK
    pltpu.sync_copy(u_hbm.at[pl.ds(base, CHUNK)], ub)
    pltpu.sync_copy(ids_hbm.at[pl.ds(base, CHUNK)], idsb.at[pl.ds(0, CHUNK)])

    lanes = lax.iota(jnp.int32, L)
    idsb[pl.ds(CHUNK, L)] = jnp.full((L,), -1, jnp.int32)

    lrb[...] = jnp.full((L,), NEG, jnp.float32)
    lo = jnp.min(idsb[pl.ds(0, L)])
    hi = jnp.max(idsb[pl.ds(CHUNK - L, L)])

    def seg_max(s, carry):
        def g_body(g, maccs):
            new = []
            for k in range(4):
                sl = pl.ds((4 * g + k) * L, L)
                new.append(jnp.maximum(
                    maccs[k], jnp.where(idsb[sl] == s, ub[sl], NEG)))
            return tuple(new)
        init = tuple(jnp.full((L,), NEG, jnp.float32) for _ in range(4))
        maccs = lax.fori_loop(0, G // 4, g_body, init)
        m = jnp.max(jnp.maximum(jnp.maximum(maccs[0], maccs[1]),
                                jnp.maximum(maccs[2], maccs[3])))
        plsc.store_scatter(lrb, [jnp.full((L,), s, jnp.int32)],
                           jnp.full((L,), m, jnp.float32), mask=lanes == 0)
        return carry

    lax.fori_loop(lo, hi + 1, seg_max, 0)

    pltpu.sync_copy(lrb, shmax.at[pl.ds(w * NSEG, NSEG)])
    plsc.subcore_barrier()
    pltpu.sync_copy(shmax, allb)
    gm = allb[pl.ds(0, L)]
    for r in range(1, NW):
        gm = jnp.maximum(gm, allb[pl.ds(r * L, L)])
    gb[...] = gm

    lrb[...] = jnp.zeros((L,), jnp.float32)

    @plsc.parallel_loop(0, G, unroll=4)
    def pass_b1(g):
        sl = pl.ds(g * L, L)
        mseg = plsc.load_gather(gb, [idsb[sl]])
        ev = jnp.exp(ub[sl] - mseg)
        ub[sl] = ev
        plsc.store_scatter(totb, [jnp.full((L,), g, jnp.int32)],
                           jnp.full((L,), jnp.sum(ev), jnp.float32),
                           mask=lanes == 0)

    def prefix(k, carry):
        sl = pl.ds(k * L, L)
        tv = totb[sl]
        pc = plsc.cumsum(tv)
        tb[pl.ds(0, L)] = pc
        sh = plsc.load_gather(tb, [jnp.maximum(lanes - 1, 0)])
        excl = jnp.where(lanes == 0, 0.0, sh)
        offsb[sl] = carry + excl
        return carry + jnp.sum(tv)

    lax.fori_loop(0, G // L, prefix, jnp.zeros((L,), jnp.float32))

    @plsc.parallel_loop(0, G, unroll=4)
    def pass_b3(g):
        sl = pl.ds(g * L, L)
        idsv = idsb[sl]
        t = plsc.cumsum(ub[sl]) + plsc.load_gather(
            offsb, [jnp.full((L,), g, jnp.int32)])
        bmask = idsv != idsb[pl.ds(g * L + 1, L)]
        plsc.store_scatter(lrb, [idsv], t, mask=bmask)

    v = lrb[...]
    fwd = plsc.cummax(v)
    tb[pl.ds(0, L)] = fwd
    prev = plsc.load_gather(tb, [jnp.maximum(lanes - 1, 0)])
    prev = jnp.where(lanes == 0, 0.0, prev)
    lrb[...] = jnp.where(v == 0.0, 0.0, v - prev)

    pltpu.sync_copy(lrb, shsum.at[pl.ds(w * NSEG, NSEG)])
    plsc.subcore_barrier()
    pltpu.sync_copy(shsum, allb)
    gs = allb[pl.ds(0, L)]
    for r in range(1, NW):
        gs = gs + allb[pl.ds(r * L, L)]
    gb[...] = 1.0 / gs

    @plsc.parallel_loop(0, G, unroll=4)
    def pass_c(g):
        sl = pl.ds(g * L, L)
        rseg = plsc.load_gather(gb, [idsb[sl]])
        ob[sl] = ub[sl] * rseg

    pltpu.sync_copy(ob, out_hbm.at[pl.ds(base, CHUNK)])


def kernel(x, ids, W):
    ids32 = ids.astype(jnp.int32)

    u = pl.pallas_call(
        _tc_matvec,
        grid=(2,),
        in_specs=[pl.BlockSpec((D, N // 2), lambda i: (0, i)),
                  pl.BlockSpec((1, D), lambda i: (0, 0))],
        out_specs=pl.BlockSpec((N // 2,), lambda i: (i,)),
        out_shape=jax.ShapeDtypeStruct((N,), jnp.float32),
    )(x.T, W)

    mesh = plsc.VectorSubcoreMesh(
        core_axis_name="c", subcore_axis_name="s", num_cores=1,
        num_subcores=NW)
    soft = pl.kernel(
        _sc_softmax,
        out_type=jax.ShapeDtypeStruct((N,), jnp.float32),
        mesh=mesh,
        compiler_params=pltpu.CompilerParams(
            needs_layout_passes=False, disable_bounds_checks=True),
        scratch_types=[
            pltpu.VMEM((CHUNK,), jnp.float32),
            pltpu.VMEM((CHUNK + L,), jnp.int32),
            pltpu.VMEM((NSEG,), jnp.float32),
            pltpu.VMEM((NW * NSEG,), jnp.float32),
            pltpu.VMEM((NSEG,), jnp.float32),
            pltpu.VMEM((CHUNK,), jnp.float32),
            pltpu.VMEM((L,), jnp.float32),
            pltpu.VMEM((G,), jnp.float32),
            pltpu.VMEM((G,), jnp.float32),
            pltpu.VMEM_SHARED((NW * NSEG,), jnp.float32),
            pltpu.VMEM_SHARED((NW * NSEG,), jnp.float32),
        ],
    )(u, ids32)
    return soft[:, None]
